# baseline (speedup 1.0000x reference)
import jax
import jax.numpy as jnp
from jax import lax
from jax.experimental import pallas as pl
from jax.experimental.pallas import tpu as pltpu
from jax.experimental.pallas import tpu_sc as plsc

SEG_LEN = 32
OUT_LEN = 1056
ZERO_ROWS = 1008
ZBLK = 2
N_CHUNKS = ZERO_ROWS // ZBLK
NC = 2
NS = 16
NW = NC * NS


def kernel(current_segment):
    seg_len, batch, emb = current_segment.shape
    mesh = plsc.VectorSubcoreMesh(core_axis_name="c", subcore_axis_name="s")

    @pl.kernel(
        mesh=mesh,
        out_type=jax.ShapeDtypeStruct(
            (OUT_LEN, batch, emb), current_segment.dtype
        ),
        scratch_types=[
            pltpu.VMEM((ZBLK, batch, emb), current_segment.dtype),
            pltpu.VMEM((batch, emb), current_segment.dtype),
            pltpu.SemaphoreType.DMA,
        ],
    )
    def sc_kernel(seg_hbm, out_hbm, zbuf, rbuf, sem):
        wid = lax.axis_index("s") * NC + lax.axis_index("c")

        zero16 = jnp.zeros((16,), dtype=zbuf.dtype)

        @pl.loop(0, ZBLK)
        def _(a):
            @pl.loop(0, batch)
            def _(b):
                @pl.loop(0, emb, step=16)
                def _(c):
                    zbuf[a, b, pl.ds(c, 16)] = zero16

        @pl.loop(wid, N_CHUNKS, step=NW)
        def _(chunk):
            pltpu.async_copy(
                zbuf, out_hbm.at[pl.ds(chunk * ZBLK, ZBLK)], sem
            ).wait()

        @pl.loop(wid, 48, step=NW)
        def _(t):
            src = jnp.where(t < 16, 2 * t, t - 16)
            pltpu.async_copy(seg_hbm.at[src], rbuf, sem).wait()
            pltpu.async_copy(rbuf, out_hbm.at[ZERO_ROWS + t], sem).wait()

    return sc_kernel(current_segment)


# speedup vs baseline: 1.9229x; 1.9229x over previous
import jax
import jax.numpy as jnp
from jax.experimental import pallas as pl
import jax.experimental.pallas.tpu as pltpu

SEG_LEN = 32
OUT_LEN = 1056
ZERO_ROWS = 1008
ZBLK = 48
NZ = ZERO_ROWS // ZBLK


def _body(seg_hbm, out_hbm, zbuf, dbuf, zsems, dsem):
    in_cp = pltpu.make_async_copy(seg_hbm, dbuf.at[pl.ds(16, SEG_LEN)], dsem)
    in_cp.start()

    zbuf[...] = jnp.zeros_like(zbuf)
    for i in range(NZ):
        pltpu.make_async_copy(
            zbuf, out_hbm.at[pl.ds(i * ZBLK, ZBLK)], zsems.at[i]
        ).start()

    in_cp.wait()
    seg = dbuf[pl.ds(16, SEG_LEN)]
    dbuf[0:16] = seg.reshape(16, 2, *seg.shape[1:])[:, 0]
    data_cp = pltpu.make_async_copy(
        dbuf, out_hbm.at[pl.ds(ZERO_ROWS, 48)], dsem
    )
    data_cp.start()

    for i in range(NZ):
        pltpu.make_async_copy(
            zbuf, out_hbm.at[pl.ds(i * ZBLK, ZBLK)], zsems.at[i]
        ).wait()
    data_cp.wait()


def kernel(current_segment):
    seg_len, batch, emb = current_segment.shape

    return pl.pallas_call(
        _body,
        in_specs=[pl.BlockSpec(memory_space=pltpu.MemorySpace.HBM)],
        out_specs=pl.BlockSpec(memory_space=pltpu.MemorySpace.HBM),
        out_shape=jax.ShapeDtypeStruct(
            (OUT_LEN, batch, emb), current_segment.dtype
        ),
        scratch_shapes=[
            pltpu.VMEM((ZBLK, batch, emb), current_segment.dtype),
            pltpu.VMEM((48, batch, emb), current_segment.dtype),
            pltpu.SemaphoreType.DMA((NZ,)),
            pltpu.SemaphoreType.DMA,
        ],
    )(current_segment)
